# baseline (speedup 1.0000x reference)
import jax
import jax.numpy as jnp
from jax import lax
from jax.experimental import pallas as pl
from jax.experimental.pallas import tpu as pltpu
from jax.experimental.pallas import tpu_sc as plsc

N = 10000
NP = 10240
D = 128
NSUB = 16
EPT = 20480
TOT = NSUB * EPT
EROWS = TOT // 128
ROWS_PT = EPT // 128
CROWS = EROWS // 32
NPT = NP // NSUB
SEG = 16


def _rsqrt_newton(x):
    i = plsc.bitcast(x, jnp.int32)
    i = jnp.int32(0x5F3759DF) - lax.shift_right_arithmetic(i, 1)
    y = plsc.bitcast(i, jnp.float32)
    for _ in range(3):
        y = y * (1.5 - 0.5 * x * y * y)
    return y


def _degree_body(pair, isd, idx_v, hist_v, red_v, sh):
    c = lax.axis_index("c")
    s = lax.axis_index("s")
    pltpu.sync_copy(pair.at[c, pl.ds(s * ROWS_PT, ROWS_PT)], idx_v)

    zeros = jnp.zeros((16,), jnp.float32)

    def zbody(k, carry):
        hist_v[pl.ds(k * 16, 16)] = zeros
        return carry

    lax.fori_loop(0, NP // 16, zbody, 0)

    ones = jnp.ones((16,), jnp.float32)

    def sbody(r, carry):
        for j in range(8):
            iv = idx_v[r, pl.ds(j * 16, 16)]
            plsc.addupdate_scatter(hist_v, [iv], ones)
        return carry

    lax.fori_loop(0, ROWS_PT, sbody, 0)

    pltpu.sync_copy(hist_v, sh.at[s])
    plsc.subcore_barrier()
    for r in range(NSUB):
        pltpu.sync_copy(sh.at[r, pl.ds(s * NPT, NPT)], red_v.at[r])

    def rbody(k, carry):
        acc = red_v[0, pl.ds(k * 16, 16)]
        for r in range(1, NSUB):
            acc = acc + red_v[r, pl.ds(k * 16, 16)]
        deg = jnp.maximum(acc, 1.0)
        hist_v[pl.ds(k * 16, 16)] = _rsqrt_newton(deg)
        return carry

    lax.fori_loop(0, NPT // 16, rbody, 0)
    pltpu.sync_copy(hist_v.at[pl.ds(0, NPT)], isd.at[c, pl.ds(s * NPT, NPT)])


def _coef_body(pair, w, isd, coef, uix_v, iix_v, cf_v, su_v, si_v):
    c = lax.axis_index("c")
    s = lax.axis_index("s")
    base = (s * 2 + c) * CROWS
    pltpu.sync_copy(pair.at[0, pl.ds(base, CROWS)], uix_v)
    pltpu.sync_copy(pair.at[1, pl.ds(base, CROWS)], iix_v)
    pltpu.sync_copy(w.at[pl.ds(base, CROWS)], cf_v)
    pltpu.sync_copy(isd.at[0], su_v)
    pltpu.sync_copy(isd.at[1], si_v)

    def cbody(r, carry):
        for j in range(8):
            sl = pl.ds(j * 16, 16)
            gu = plsc.load_gather(su_v, [uix_v[r, sl]])
            gi = plsc.load_gather(si_v, [iix_v[r, sl]])
            cf_v[r, sl] = cf_v[r, sl] * gu * gi
        return carry

    lax.fori_loop(0, CROWS, cbody, 0)
    pltpu.sync_copy(cf_v, coef.at[pl.ds(base, CROWS)])


def _agg_body(table, pair, ssrc, coef, out, six_v, dix_v, cf_v, rows0_v,
              rows1_v, gsem0, gsem1, ssem0, ssem1, acc_sh):
    c = lax.axis_index("c")
    s = lax.axis_index("s")
    rows = (rows0_v, rows1_v)
    gsem = (gsem0, gsem1)
    ssem = (ssem0, ssem1)

    zeros = jnp.zeros((16,), jnp.float32)

    def zbody(r, carry):
        for j in range(8):
            rows0_v[r, pl.ds(j * 16, 16)] = zeros
        return carry

    lax.fori_loop(0, 128, zbody, 0)
    for k in range(NPT // 128):
        pltpu.sync_copy(rows0_v, acc_sh.at[pl.ds(s * NPT + k * 128, 128)])
    plsc.subcore_barrier()

    def wait_gather(p):
        pltpu.make_async_copy(table.at[six_v.at[0]], rows[p], gsem[p]).wait()

    def wait_scatter(p):
        pltpu.make_async_copy(rows[p], acc_sh.at[dix_v.at[0]], ssem[p]).wait()

    def scale_batch(b, p):
        def scale(g, carry):
            cvec = cf_v[b, pl.ds(g * 16, 16)]
            for i in range(16):
                cv = cvec.at[jnp.full((16,), i, jnp.int32)].get(
                    mode="promise_in_bounds")
                rr = g * 16 + i
                for j in range(8):
                    sl = pl.ds(j * 16, 16)
                    rows[p][rr, sl] = rows[p][rr, sl] * cv
            return carry

        lax.fori_loop(0, 8, scale, 0)

    def gbody(g, carry):
        row0 = s * ROWS_PT + g * SEG
        pltpu.sync_copy(ssrc.at[c, pl.ds(row0, SEG)], six_v)
        pltpu.sync_copy(pair.at[c, pl.ds(row0, SEG)], dix_v)
        pltpu.sync_copy(coef.at[pl.ds(row0, SEG)], cf_v)

        pltpu.async_copy(table.at[six_v.at[0]], rows[0], gsem[0])
        wait_gather(0)
        pltpu.async_copy(table.at[six_v.at[1]], rows[1], gsem[1])
        scale_batch(0, 0)
        pltpu.async_copy(rows[0], acc_sh.at[dix_v.at[0]], ssem[0], add=True)

        def pbody(k, carry2):
            b1 = 2 * k + 1
            wait_gather(1)
            wait_scatter(0)
            pltpu.async_copy(table.at[six_v.at[b1 + 1]], rows[0], gsem[0])
            scale_batch(b1, 1)
            pltpu.async_copy(rows[1], acc_sh.at[dix_v.at[b1]], ssem[1],
                             add=True)
            b2 = 2 * k + 2
            wait_gather(0)
            wait_scatter(1)
            pltpu.async_copy(table.at[six_v.at[b2 + 1]], rows[1], gsem[1])
            scale_batch(b2, 0)
            pltpu.async_copy(rows[0], acc_sh.at[dix_v.at[b2]], ssem[0],
                             add=True)
            return carry2

        lax.fori_loop(0, SEG // 2 - 1, pbody, 0)
        wait_gather(1)
        wait_scatter(0)
        scale_batch(SEG - 1, 1)
        pltpu.async_copy(rows[1], acc_sh.at[dix_v.at[SEG - 1]], ssem[1],
                         add=True)
        wait_scatter(1)
        return carry

    lax.fori_loop(0, ROWS_PT // SEG, gbody, 0)
    plsc.subcore_barrier()

    for k in range(NPT // 128):
        sl = pl.ds(s * NPT + k * 128, 128)
        pltpu.sync_copy(acc_sh.at[sl], rows0_v)
        pltpu.sync_copy(rows0_v, out.at[c, sl])


_mesh = plsc.VectorSubcoreMesh(
    core_axis_name="c", subcore_axis_name="s", num_cores=2, num_subcores=NSUB)

_params = pltpu.CompilerParams(needs_layout_passes=False)

_degree_kernel = pl.kernel(
    _degree_body,
    compiler_params=_params,
    out_type=jax.ShapeDtypeStruct((2, NP), jnp.float32),
    mesh=_mesh,
    scratch_types=[
        pltpu.VMEM((ROWS_PT, 128), jnp.int32),
        pltpu.VMEM((NP,), jnp.float32),
        pltpu.VMEM((NSUB, NPT), jnp.float32),
        pltpu.VMEM_SHARED((NSUB, NP), jnp.float32),
    ],
)

_coef_kernel = pl.kernel(
    _coef_body,
    compiler_params=_params,
    out_type=jax.ShapeDtypeStruct((EROWS, 128), jnp.float32),
    mesh=_mesh,
    scratch_types=[
        pltpu.VMEM((CROWS, 128), jnp.int32),
        pltpu.VMEM((CROWS, 128), jnp.int32),
        pltpu.VMEM((CROWS, 128), jnp.float32),
        pltpu.VMEM((NP,), jnp.float32),
        pltpu.VMEM((NP,), jnp.float32),
    ],
)

_agg_kernel = pl.kernel(
    _agg_body,
    compiler_params=_params,
    out_type=jax.ShapeDtypeStruct((2, NP, D), jnp.float32),
    mesh=_mesh,
    scratch_types=[
        pltpu.VMEM((SEG, 128), jnp.int32),
        pltpu.VMEM((SEG, 128), jnp.int32),
        pltpu.VMEM((SEG, 128), jnp.float32),
        pltpu.VMEM((128, D), jnp.float32),
        pltpu.VMEM((128, D), jnp.float32),
        pltpu.SemaphoreType.DMA,
        pltpu.SemaphoreType.DMA,
        pltpu.SemaphoreType.DMA,
        pltpu.SemaphoreType.DMA,
        pltpu.VMEM_SHARED((NP, D), jnp.float32),
    ],
)


def kernel(u_emb, i_emb, edge_index, weights):
    user_idx = edge_index[0].astype(jnp.int32)
    item_idx = edge_index[1].astype(jnp.int32)
    e = weights.shape[0]
    pad = TOT - e
    fill = jnp.full((pad,), N, jnp.int32)
    pair = jnp.stack([
        jnp.concatenate([user_idx, fill]),
        jnp.concatenate([item_idx, fill]),
    ]).reshape(2, EROWS, 128)
    w = jnp.concatenate(
        [weights, jnp.zeros((pad,), jnp.float32)]).reshape(EROWS, 128)
    table = jnp.concatenate([
        jnp.pad(i_emb, ((0, NP - N), (0, 0))),
        jnp.pad(u_emb, ((0, NP - N), (0, 0))),
    ])
    ssrc = jnp.stack([pair[1], pair[0] + NP])
    isd = _degree_kernel(pair)
    coef = _coef_kernel(pair, w, isd)
    out = _agg_kernel(table, pair, ssrc, coef)
    return out[0, :N], out[1, :N]


# speedup vs baseline: 1.0992x; 1.0992x over previous
import jax
import jax.numpy as jnp
from jax import lax
from jax.experimental import pallas as pl
from jax.experimental.pallas import tpu as pltpu
from jax.experimental.pallas import tpu_sc as plsc

N = 10000
NP = 10240
D = 128
NSUB = 16
EPT = 20480
TOT = NSUB * EPT
EROWS = TOT // 128
ROWS_PT = EPT // 128
CROWS = EROWS // 32
NPT = NP // NSUB
SEG = 32


def _rsqrt_newton(x):
    i = plsc.bitcast(x, jnp.int32)
    i = jnp.int32(0x5F3759DF) - lax.shift_right_arithmetic(i, 1)
    y = plsc.bitcast(i, jnp.float32)
    for _ in range(3):
        y = y * (1.5 - 0.5 * x * y * y)
    return y


def _degree_body(pair, isd, idx_v, hist_v, red_v, sh):
    c = lax.axis_index("c")
    s = lax.axis_index("s")
    pltpu.sync_copy(pair.at[c, pl.ds(s * ROWS_PT, ROWS_PT)], idx_v)

    zeros = jnp.zeros((16,), jnp.float32)

    def zbody(k, carry):
        hist_v[pl.ds(k * 16, 16)] = zeros
        return carry

    lax.fori_loop(0, NP // 16, zbody, 0)

    ones = jnp.ones((16,), jnp.float32)

    def sbody(r, carry):
        for j in range(8):
            iv = idx_v[r, pl.ds(j * 16, 16)]
            plsc.addupdate_scatter(hist_v, [iv], ones)
        return carry

    lax.fori_loop(0, ROWS_PT, sbody, 0)

    pltpu.sync_copy(hist_v, sh.at[s])
    plsc.subcore_barrier()
    for r in range(NSUB):
        pltpu.sync_copy(sh.at[r, pl.ds(s * NPT, NPT)], red_v.at[r])

    def rbody(k, carry):
        acc = red_v[0, pl.ds(k * 16, 16)]
        for r in range(1, NSUB):
            acc = acc + red_v[r, pl.ds(k * 16, 16)]
        deg = jnp.maximum(acc, 1.0)
        hist_v[pl.ds(k * 16, 16)] = _rsqrt_newton(deg)
        return carry

    lax.fori_loop(0, NPT // 16, rbody, 0)
    pltpu.sync_copy(hist_v.at[pl.ds(0, NPT)], isd.at[c, pl.ds(s * NPT, NPT)])


def _coef_body(pair, w, isd, coef, uix_v, iix_v, cf_v, su_v, si_v):
    c = lax.axis_index("c")
    s = lax.axis_index("s")
    base = (s * 2 + c) * CROWS
    pltpu.sync_copy(pair.at[0, pl.ds(base, CROWS)], uix_v)
    pltpu.sync_copy(pair.at[1, pl.ds(base, CROWS)], iix_v)
    pltpu.sync_copy(w.at[pl.ds(base, CROWS)], cf_v)
    pltpu.sync_copy(isd.at[0], su_v)
    pltpu.sync_copy(isd.at[1], si_v)

    def cbody(r, carry):
        for j in range(8):
            sl = pl.ds(j * 16, 16)
            gu = plsc.load_gather(su_v, [uix_v[r, sl]])
            gi = plsc.load_gather(si_v, [iix_v[r, sl]])
            cf_v[r, sl] = cf_v[r, sl] * gu * gi
        return carry

    lax.fori_loop(0, CROWS, cbody, 0)
    pltpu.sync_copy(cf_v, coef.at[pl.ds(base, CROWS)])


def _agg_body(table, pair, coef, out, six_v, dix_v, cf_v, rows0_v, rows1_v,
              gsem0, gsem1, ssem0, ssem1, acc_sh):
    c = lax.axis_index("c")
    s = lax.axis_index("s")
    rows = (rows0_v, rows1_v)
    gsem = (gsem0, gsem1)
    ssem = (ssem0, ssem1)

    zeros = jnp.zeros((16,), jnp.float32)

    def zbody(r, carry):
        for j in range(8):
            rows0_v[r, pl.ds(j * 16, 16)] = zeros
        return carry

    lax.fori_loop(0, 128, zbody, 0)
    for k in range(NPT // 128):
        pltpu.sync_copy(rows0_v, acc_sh.at[pl.ds(s * NPT + k * 128, 128)])
    plsc.subcore_barrier()

    src_off = c * NP

    def wait_gather(p):
        pltpu.make_async_copy(table.at[six_v.at[0]], rows[p], gsem[p]).wait()

    def wait_scatter(p):
        pltpu.make_async_copy(rows[p], acc_sh.at[dix_v.at[0]], ssem[p]).wait()

    def scale_batch(b, p):
        @plsc.parallel_loop(0, 8, unroll=2)
        def scale(g):
            cvec = cf_v[b, pl.ds(g * 16, 16)]
            for i in range(16):
                cv = cvec.at[jnp.full((16,), i, jnp.int32)].get(
                    mode="promise_in_bounds")
                rr = g * 16 + i
                for j in range(8):
                    sl = pl.ds(j * 16, 16)
                    rows[p][rr, sl] = rows[p][rr, sl] * cv

    def gbody(g, carry):
        row0 = s * ROWS_PT + g * SEG
        pltpu.sync_copy(pair.at[1 - c, pl.ds(row0, SEG)], six_v)
        pltpu.sync_copy(pair.at[c, pl.ds(row0, SEG)], dix_v)
        pltpu.sync_copy(coef.at[pl.ds(row0, SEG)], cf_v)

        @plsc.parallel_loop(0, SEG, unroll=2)
        def obody(r):
            for j in range(8):
                sl = pl.ds(j * 16, 16)
                six_v[r, sl] = six_v[r, sl] + src_off

        pltpu.async_copy(table.at[six_v.at[0]], rows[0], gsem[0])
        wait_gather(0)
        pltpu.async_copy(table.at[six_v.at[1]], rows[1], gsem[1])
        scale_batch(0, 0)
        pltpu.async_copy(rows[0], acc_sh.at[dix_v.at[0]], ssem[0], add=True)

        def pbody(k, carry2):
            b1 = 2 * k + 1
            wait_gather(1)
            wait_scatter(0)
            pltpu.async_copy(table.at[six_v.at[b1 + 1]], rows[0], gsem[0])
            scale_batch(b1, 1)
            pltpu.async_copy(rows[1], acc_sh.at[dix_v.at[b1]], ssem[1],
                             add=True)
            b2 = 2 * k + 2
            wait_gather(0)
            wait_scatter(1)
            pltpu.async_copy(table.at[six_v.at[b2 + 1]], rows[1], gsem[1])
            scale_batch(b2, 0)
            pltpu.async_copy(rows[0], acc_sh.at[dix_v.at[b2]], ssem[0],
                             add=True)
            return carry2

        lax.fori_loop(0, SEG // 2 - 1, pbody, 0)
        wait_gather(1)
        wait_scatter(0)
        scale_batch(SEG - 1, 1)
        pltpu.async_copy(rows[1], acc_sh.at[dix_v.at[SEG - 1]], ssem[1],
                         add=True)
        wait_scatter(1)
        return carry

    lax.fori_loop(0, ROWS_PT // SEG, gbody, 0)
    plsc.subcore_barrier()

    sl = pl.ds(s * NPT, NPT)
    pltpu.sync_copy(acc_sh.at[sl], out.at[c, sl])


_mesh = plsc.VectorSubcoreMesh(
    core_axis_name="c", subcore_axis_name="s", num_cores=2, num_subcores=NSUB)

_params = pltpu.CompilerParams(needs_layout_passes=False)

_degree_kernel = pl.kernel(
    _degree_body,
    compiler_params=_params,
    out_type=jax.ShapeDtypeStruct((2, NP), jnp.float32),
    mesh=_mesh,
    scratch_types=[
        pltpu.VMEM((ROWS_PT, 128), jnp.int32),
        pltpu.VMEM((NP,), jnp.float32),
        pltpu.VMEM((NSUB, NPT), jnp.float32),
        pltpu.VMEM_SHARED((NSUB, NP), jnp.float32),
    ],
)

_coef_kernel = pl.kernel(
    _coef_body,
    compiler_params=_params,
    out_type=jax.ShapeDtypeStruct((EROWS, 128), jnp.float32),
    mesh=_mesh,
    scratch_types=[
        pltpu.VMEM((CROWS, 128), jnp.int32),
        pltpu.VMEM((CROWS, 128), jnp.int32),
        pltpu.VMEM((CROWS, 128), jnp.float32),
        pltpu.VMEM((NP,), jnp.float32),
        pltpu.VMEM((NP,), jnp.float32),
    ],
)

_agg_kernel = pl.kernel(
    _agg_body,
    compiler_params=_params,
    out_type=jax.ShapeDtypeStruct((2, NP, D), jnp.float32),
    mesh=_mesh,
    scratch_types=[
        pltpu.VMEM((SEG, 128), jnp.int32),
        pltpu.VMEM((SEG, 128), jnp.int32),
        pltpu.VMEM((SEG, 128), jnp.float32),
        pltpu.VMEM((128, D), jnp.float32),
        pltpu.VMEM((128, D), jnp.float32),
        pltpu.SemaphoreType.DMA,
        pltpu.SemaphoreType.DMA,
        pltpu.SemaphoreType.DMA,
        pltpu.SemaphoreType.DMA,
        pltpu.VMEM_SHARED((NP, D), jnp.float32),
    ],
)


def kernel(u_emb, i_emb, edge_index, weights):
    user_idx = edge_index[0].astype(jnp.int32)
    item_idx = edge_index[1].astype(jnp.int32)
    e = weights.shape[0]
    pad = TOT - e
    fill = jnp.full((pad,), N, jnp.int32)
    pair = jnp.stack([
        jnp.concatenate([user_idx, fill]),
        jnp.concatenate([item_idx, fill]),
    ]).reshape(2, EROWS, 128)
    w = jnp.concatenate(
        [weights, jnp.zeros((pad,), jnp.float32)]).reshape(EROWS, 128)
    table = jnp.concatenate([
        jnp.pad(i_emb, ((0, NP - N), (0, 0))),
        jnp.pad(u_emb, ((0, NP - N), (0, 0))),
    ])
    isd = _degree_kernel(pair)
    coef = _coef_kernel(pair, w, isd)
    out = _agg_kernel(table, pair, coef)
    return out[0, :N], out[1, :N]
